# baseline (speedup 1.0000x reference)
import jax
import jax.numpy as jnp
import numpy as np
from jax import lax
from jax.experimental import pallas as pl

IN_CH = 96
HID = 192
NUM_CLASSES = 80
OUT_ROWS = 88
GN_EPS = 1e-05
STRIDES = (8, 16, 32, 64, 128)
SIZES = ((64, 64), (32, 32), (16, 16), (8, 8), (4, 4))
NLEV = 5
KSPLIT = 4
IO = ((32, 128), (1024,), (256,), (64,), (16,))


def _lane_sum(v):
    R, S = v.shape
    if S >= 128 and S % 128 == 0:
        p = jnp.sum(v.reshape(R, S // 128, 128), axis=1)
        return jnp.sum(p, axis=1, keepdims=True)
    return jnp.sum(v, axis=1, keepdims=True)


def _do_level(x_ref, w3_ref, params, wf_ref, fb_ref,
              cls_ref, cent_ref, reg_ref, H, W, stride):
    b3, gam, bet = params
    S = H * W
    logw = W.bit_length() - 1
    x = x_ref[0].astype(jnp.bfloat16).reshape(IN_CH, S)
    pos = lax.broadcasted_iota(jnp.int32, (1, S), 1)
    col = pos & (W - 1)
    row = pos >> logw
    row_ok = {dy: (row + dy >= 0) & (row + dy < H) for dy in (-1, 0, 1)}
    col_ok = {dx: (col + dx >= 0) & (col + dx < W) for dx in (-1, 0, 1)}
    parts = []
    for dy in (-1, 0, 1):
        for dx in (-1, 0, 1):
            k = dy * W + dx
            sh = jnp.roll(x, -k, axis=1) if k else x
            parts.append(jnp.where(row_ok[dy] & col_ok[dx], sh,
                                   jnp.bfloat16(0)))
    xcol_a = jnp.concatenate(parts[:KSPLIT], axis=0)
    xcol_b = jnp.concatenate(parts[KSPLIT:], axis=0)
    ka = KSPLIT * IN_CH
    dn = (((1,), (0,)), ((), ()))
    h = lax.dot_general(w3_ref[:, 0:ka], xcol_a, dn,
                        preferred_element_type=jnp.float32)
    h = h + lax.dot_general(w3_ref[:, ka:9 * IN_CH], xcol_b, dn,
                            preferred_element_type=jnp.float32)
    h = h + b3
    s1 = _lane_sum(h)
    s2 = _lane_sum(h * h)
    st = jnp.concatenate([s1, s2], axis=0)
    ci = lax.broadcasted_iota(jnp.int32, (2 * HID, 1), 0)
    a1 = st + jnp.roll(st, -1, axis=0) + jnp.roll(st, -2, axis=0)
    g0 = jnp.where(ci % 3 == 0, a1, 0.0)
    gs = g0 + jnp.roll(g0, 1, axis=0) + jnp.roll(g0, 2, axis=0)
    cnt = 1.0 / (3.0 * S)
    mean = gs[0:HID] * cnt
    var = gs[HID:2 * HID] * cnt - mean * mean
    inv = lax.rsqrt(var + GN_EPS)
    scale = inv * gam
    shift = bet - mean * scale
    hn = h * scale + shift
    a = hn * jax.nn.sigmoid(hn)
    y = lax.dot_general(wf_ref[...], a.astype(jnp.bfloat16), dn,
                        preferred_element_type=jnp.float32)
    y = y + fb_ref[...]
    io = cls_ref.shape[2:]
    cls_ref[0] = y[0:NUM_CLASSES].reshape((NUM_CLASSES,) + io)
    rc = y[NUM_CLASSES:NUM_CLASSES + 8]
    ri = lax.broadcasted_iota(jnp.int32, (8, 1), 0)
    rcp = jnp.where((ri >= 1) & (ri < 5),
                    jnp.maximum(rc * jnp.float32(stride), 0.0), rc)
    cent_ref[0] = rcp[0:1].reshape((1,) + io)
    reg_ref[0] = rcp[1:5].reshape((4,) + io)


def _fused_kernel(*refs):
    x_refs = refs[0:NLEV]
    w3_ref, pm_ref, wf_ref, fb_ref = refs[NLEV:NLEV + 4]
    cls_refs = refs[NLEV + 4:NLEV + 4 + NLEV]
    reg_refs = refs[NLEV + 4 + NLEV:NLEV + 4 + 2 * NLEV]
    cent_refs = refs[NLEV + 4 + 2 * NLEV:]
    pm = pm_ref[...]
    b3 = jnp.concatenate([pm[:, 0:1], pm[:, 3:4]], axis=0)
    gam = jnp.concatenate([pm[:, 1:2], pm[:, 4:5]], axis=0)
    bet = jnp.concatenate([pm[:, 2:3], pm[:, 5:6]], axis=0)
    params = (b3, gam, bet)
    for l in range(NLEV):
        H, W = SIZES[l]
        _do_level(x_refs[l], w3_ref, params, wf_ref, fb_ref,
                  cls_refs[l], cent_refs[l], reg_refs[l],
                  H, W, STRIDES[l])


def kernel(fpn0, fpn1, fpn2, fpn3, fpn4,
           cls_w, cls_b, cls_g, cls_beta, cls_fw, cls_fb,
           reg_w, reg_b, reg_g, reg_beta, reg_fw, reg_fb):
    f32 = jnp.float32
    B = fpn0.shape[0]
    wtap = jnp.concatenate([cls_w, reg_w], axis=0)
    W3 = jnp.transpose(wtap, (0, 2, 3, 1)).reshape(HID, 9 * IN_CH)
    W3 = W3.astype(jnp.bfloat16)
    pm = jnp.stack([cls_b, cls_g, cls_beta, reg_b, reg_g, reg_beta,
                    jnp.zeros_like(cls_b), jnp.zeros_like(cls_b)], axis=1)
    Wf = jnp.zeros((OUT_ROWS, HID), f32)
    Wf = Wf.at[0:NUM_CLASSES, 0:IN_CH].set(cls_fw[:, :, 0, 0])
    Wf = Wf.at[NUM_CLASSES:NUM_CLASSES + 5, IN_CH:HID].set(reg_fw[:, :, 0, 0])
    Wf = Wf.astype(jnp.bfloat16)
    fb = jnp.concatenate([cls_fb, reg_fb,
                          jnp.zeros((3,), f32)]).reshape(OUT_ROWS, 1)

    fpns = (fpn0, fpn1, fpn2, fpn3, fpn4)
    xs = [fpns[0].reshape((B, IN_CH) + IO[0])] + [
        x.astype(jnp.bfloat16).reshape((B, IN_CH) + IO[l])
        for l, x in enumerate(fpns) if l > 0]
    full = lambda shp: pl.BlockSpec(shp, lambda b: (0,) * len(shp))
    bspec = lambda c, io: pl.BlockSpec((1, c) + io,
                                       lambda b: (b,) + (0,) * (1 + len(io)))
    out_shape = (
        tuple(jax.ShapeDtypeStruct((B, NUM_CLASSES) + io, f32) for io in IO)
        + tuple(jax.ShapeDtypeStruct((B, 4) + io, f32) for io in IO)
        + tuple(jax.ShapeDtypeStruct((B, 1) + io, f32) for io in IO)
    )
    outs = pl.pallas_call(
        _fused_kernel,
        grid=(B,),
        in_specs=(
            [bspec(IN_CH, io) for io in IO]
            + [full((HID, 9 * IN_CH)),
               full((IN_CH, 8)), full((OUT_ROWS, HID)), full((OUT_ROWS, 1))]
        ),
        out_specs=(
            tuple(bspec(NUM_CLASSES, io) for io in IO)
            + tuple(bspec(4, io) for io in IO)
            + tuple(bspec(1, io) for io in IO)
        ),
        out_shape=out_shape,
    )(*xs, W3, pm, Wf, fb)
    cls_out = [o.reshape(B, NUM_CLASSES, h, w)
               for o, (h, w) in zip(outs[0:NLEV], SIZES)]
    reg_out = [o.reshape(B, 4, h, w)
               for o, (h, w) in zip(outs[NLEV:2 * NLEV], SIZES)]
    cent_out = [o.reshape(B, 1, h, w)
                for o, (h, w) in zip(outs[2 * NLEV:], SIZES)]
    return tuple(cls_out) + tuple(reg_out) + tuple(cent_out)


# speedup vs baseline: 1.0571x; 1.0571x over previous
import jax
import jax.numpy as jnp
import numpy as np
from jax import lax
from jax.experimental import pallas as pl

IN_CH = 96
HID = 192
NUM_CLASSES = 80
OUT_ROWS = 88
GN_EPS = 1e-05
STRIDES = (8, 16, 32, 64, 128)
SIZES = ((64, 64), (32, 32), (16, 16), (8, 8), (4, 4))
NLEV = 5
KSPLIT = 4
CAT_OFF = (0, 1024, 1280, 1408)
CAT_LEN = 1424
IO = ((4096,), (1024,), (256,), (64,), (16,))


def _lane_sum(v):
    R, S = v.shape
    if S >= 128 and S % 128 == 0:
        p = jnp.sum(v.reshape(R, S // 128, 128), axis=1)
        return jnp.sum(p, axis=1, keepdims=True)
    return jnp.sum(v, axis=1, keepdims=True)


def _do_level(x, w3_ref, params, wf_ref, fb_ref,
              cls_ref, cent_ref, reg_ref, H, W, stride):
    b3, gam, bet = params
    S = H * W
    logw = W.bit_length() - 1
    pos = lax.broadcasted_iota(jnp.int32, (1, S), 1)
    col = pos & (W - 1)
    row = pos >> logw
    row_ok = {dy: (row + dy >= 0) & (row + dy < H) for dy in (-1, 0, 1)}
    col_ok = {dx: (col + dx >= 0) & (col + dx < W) for dx in (-1, 0, 1)}
    parts = []
    for dy in (-1, 0, 1):
        for dx in (-1, 0, 1):
            k = dy * W + dx
            sh = jnp.roll(x, -k, axis=1) if k else x
            parts.append(jnp.where(row_ok[dy] & col_ok[dx], sh,
                                   jnp.bfloat16(0)))
    xcol_a = jnp.concatenate(parts[:KSPLIT], axis=0)
    xcol_b = jnp.concatenate(parts[KSPLIT:], axis=0)
    ka = KSPLIT * IN_CH
    dn = (((1,), (0,)), ((), ()))
    h = lax.dot_general(w3_ref[:, 0:ka], xcol_a, dn,
                        preferred_element_type=jnp.float32)
    h = h + lax.dot_general(w3_ref[:, ka:9 * IN_CH], xcol_b, dn,
                            preferred_element_type=jnp.float32)
    h = h + b3
    s1 = _lane_sum(h)
    s2 = _lane_sum(h * h)
    st = jnp.concatenate([s1, s2], axis=0)
    ci = lax.broadcasted_iota(jnp.int32, (2 * HID, 1), 0)
    a1 = st + jnp.roll(st, -1, axis=0) + jnp.roll(st, -2, axis=0)
    g0 = jnp.where(ci % 3 == 0, a1, 0.0)
    gs = g0 + jnp.roll(g0, 1, axis=0) + jnp.roll(g0, 2, axis=0)
    cnt = 1.0 / (3.0 * S)
    mean = gs[0:HID] * cnt
    var = gs[HID:2 * HID] * cnt - mean * mean
    inv = lax.rsqrt(var + GN_EPS)
    scale = inv * gam
    shift = bet - mean * scale
    hn = h * scale + shift
    a = hn * (0.5 * jnp.tanh(hn * 0.5) + 0.5)
    y = lax.dot_general(wf_ref[...], a.astype(jnp.bfloat16), dn,
                        preferred_element_type=jnp.float32)
    y = y + fb_ref[...]
    io = cls_ref.shape[2:]
    cls_ref[0] = y[0:NUM_CLASSES].reshape((NUM_CLASSES,) + io)
    rc = y[NUM_CLASSES:NUM_CLASSES + 8]
    ri = lax.broadcasted_iota(jnp.int32, (8, 1), 0)
    rcp = jnp.where((ri >= 1) & (ri < 5),
                    jnp.maximum(rc * jnp.float32(stride), 0.0), rc)
    cent_ref[0] = rcp[0:1].reshape((1,) + io)
    reg_ref[0] = rcp[1:5].reshape((4,) + io)


def _fused_kernel(x0_ref, xc_ref, w3_ref, pm_ref, wf_ref, fb_ref, *out_refs):
    cls_refs = out_refs[0:NLEV]
    reg_refs = out_refs[NLEV:2 * NLEV]
    cent_refs = out_refs[2 * NLEV:]
    pm = pm_ref[...]
    b3 = jnp.concatenate([pm[:, 0:1], pm[:, 3:4]], axis=0)
    gam = jnp.concatenate([pm[:, 1:2], pm[:, 4:5]], axis=0)
    bet = jnp.concatenate([pm[:, 2:3], pm[:, 5:6]], axis=0)
    params = (b3, gam, bet)
    x0 = x0_ref[0].astype(jnp.bfloat16)
    xc = xc_ref[0]
    for l in range(NLEV):
        H, W = SIZES[l]
        if l == 0:
            x = x0
        else:
            off = CAT_OFF[l - 1]
            x = xc[:, off:off + H * W]
        _do_level(x, w3_ref, params, wf_ref, fb_ref,
                  cls_refs[l], cent_refs[l], reg_refs[l],
                  H, W, STRIDES[l])


def kernel(fpn0, fpn1, fpn2, fpn3, fpn4,
           cls_w, cls_b, cls_g, cls_beta, cls_fw, cls_fb,
           reg_w, reg_b, reg_g, reg_beta, reg_fw, reg_fb):
    f32 = jnp.float32
    B = fpn0.shape[0]
    wtap = jnp.concatenate([cls_w, reg_w], axis=0)
    W3 = jnp.transpose(wtap, (0, 2, 3, 1)).reshape(HID, 9 * IN_CH)
    W3 = W3.astype(jnp.bfloat16)
    pm = jnp.stack([cls_b, cls_g, cls_beta, reg_b, reg_g, reg_beta,
                    jnp.zeros_like(cls_b), jnp.zeros_like(cls_b)], axis=1)
    Wf = jnp.zeros((OUT_ROWS, HID), f32)
    Wf = Wf.at[0:NUM_CLASSES, 0:IN_CH].set(cls_fw[:, :, 0, 0])
    Wf = Wf.at[NUM_CLASSES:NUM_CLASSES + 5, IN_CH:HID].set(reg_fw[:, :, 0, 0])
    Wf = Wf.astype(jnp.bfloat16)
    fb = jnp.concatenate([cls_fb, reg_fb,
                          jnp.zeros((3,), f32)]).reshape(OUT_ROWS, 1)

    x0 = fpn0.reshape(B, IN_CH, 4096)
    bf16 = jnp.bfloat16
    xc = jnp.concatenate(
        [fpn1.reshape(B, IN_CH, 1024).astype(bf16),
         fpn2.reshape(B, IN_CH, 256).astype(bf16),
         fpn3.reshape(B, IN_CH, 64).astype(bf16),
         jnp.zeros((B, IN_CH, 64), bf16),
         fpn4.reshape(B, IN_CH, 16).astype(bf16)], axis=2)
    full = lambda shp: pl.BlockSpec(shp, lambda b: (0,) * len(shp))
    bspec = lambda c, io: pl.BlockSpec((1, c) + io,
                                       lambda b: (b,) + (0,) * (1 + len(io)))
    out_shape = (
        tuple(jax.ShapeDtypeStruct((B, NUM_CLASSES) + io, f32) for io in IO)
        + tuple(jax.ShapeDtypeStruct((B, 4) + io, f32) for io in IO)
        + tuple(jax.ShapeDtypeStruct((B, 1) + io, f32) for io in IO)
    )
    outs = pl.pallas_call(
        _fused_kernel,
        grid=(B,),
        in_specs=(
            [bspec(IN_CH, (4096,)), bspec(IN_CH, (CAT_LEN,)),
             full((HID, 9 * IN_CH)),
             full((IN_CH, 8)), full((OUT_ROWS, HID)), full((OUT_ROWS, 1))]
        ),
        out_specs=(
            tuple(bspec(NUM_CLASSES, io) for io in IO)
            + tuple(bspec(4, io) for io in IO)
            + tuple(bspec(1, io) for io in IO)
        ),
        out_shape=out_shape,
    )(x0, xc, W3, pm, Wf, fb)
    cls_out = [o.reshape(B, NUM_CLASSES, h, w)
               for o, (h, w) in zip(outs[0:NLEV], SIZES)]
    reg_out = [o.reshape(B, 4, h, w)
               for o, (h, w) in zip(outs[NLEV:2 * NLEV], SIZES)]
    cent_out = [o.reshape(B, 1, h, w)
                for o, (h, w) in zip(outs[2 * NLEV:], SIZES)]
    return tuple(cls_out) + tuple(reg_out) + tuple(cent_out)


# speedup vs baseline: 1.1924x; 1.1280x over previous
import jax
import jax.numpy as jnp
import numpy as np
from jax import lax
from jax.experimental import pallas as pl

IN_CH = 96
HID = 192
NUM_CLASSES = 80
OUT_ROWS = 88
GN_EPS = 1e-05
STRIDES = (8, 16, 32, 64, 128)
SIZES = ((64, 64), (32, 32), (16, 16), (8, 8), (4, 4))
NLEV = 5
KSPLIT = 4
CAT_OFF = (0, 1024, 1280, 1408)
CAT_LEN = 1424
IO = ((4096,), (1024,), (256,), (64,), (16,))


def _lane_sum(v, square=False):
    R, S = v.shape
    if S >= 256 and S % 128 == 0:
        chunks = [v[:, 128 * i:128 * (i + 1)] for i in range(S // 128)]
        if square:
            chunks = [c * c for c in chunks]
        while len(chunks) > 1:
            nxt = [chunks[j] + chunks[j + 1]
                   for j in range(0, len(chunks) - 1, 2)]
            if len(chunks) % 2:
                nxt.append(chunks[-1])
            chunks = nxt
        return jnp.sum(chunks[0], axis=1, keepdims=True)
    if square:
        v = v * v
    return jnp.sum(v, axis=1, keepdims=True)


def _do_level(x, w3_ref, params, wf_ref, fb_ref,
              cls_ref, cent_ref, reg_ref, H, W, stride):
    b3, gam, bet = params
    S = H * W
    logw = W.bit_length() - 1
    pos = lax.broadcasted_iota(jnp.int32, (1, S), 1)
    col = pos & (W - 1)
    row = pos >> logw
    row_ok = {dy: (row + dy >= 0) & (row + dy < H) for dy in (-1, 0, 1)}
    col_ok = {dx: (col + dx >= 0) & (col + dx < W) for dx in (-1, 0, 1)}
    parts = []
    for dy in (-1, 0, 1):
        for dx in (-1, 0, 1):
            k = dy * W + dx
            sh = jnp.roll(x, -k, axis=1) if k else x
            parts.append(jnp.where(row_ok[dy] & col_ok[dx], sh,
                                   jnp.bfloat16(0)))
    xcol_a = jnp.concatenate(parts[:KSPLIT], axis=0)
    xcol_b = jnp.concatenate(parts[KSPLIT:], axis=0)
    ka = KSPLIT * IN_CH
    dn = (((1,), (0,)), ((), ()))
    h = lax.dot_general(w3_ref[:, 0:ka], xcol_a, dn,
                        preferred_element_type=jnp.float32)
    h = h + lax.dot_general(w3_ref[:, ka:9 * IN_CH], xcol_b, dn,
                            preferred_element_type=jnp.float32)
    h = h + b3
    s1 = _lane_sum(h)
    s2 = _lane_sum(h, square=True)
    st = jnp.concatenate([s1, s2], axis=0)
    ci = lax.broadcasted_iota(jnp.int32, (2 * HID, 1), 0)
    a1 = st + jnp.roll(st, -1, axis=0) + jnp.roll(st, -2, axis=0)
    g0 = jnp.where(ci % 3 == 0, a1, 0.0)
    gs = g0 + jnp.roll(g0, 1, axis=0) + jnp.roll(g0, 2, axis=0)
    cnt = 1.0 / (3.0 * S)
    mean = gs[0:HID] * cnt
    var = gs[HID:2 * HID] * cnt - mean * mean
    inv = lax.rsqrt(var + GN_EPS)
    scale = inv * gam
    shift = bet - mean * scale
    hn = h * scale + shift
    a = hn * (0.5 * jnp.tanh(hn * 0.5) + 0.5)
    y = lax.dot_general(wf_ref[...], a.astype(jnp.bfloat16), dn,
                        preferred_element_type=jnp.float32)
    y = y + fb_ref[...]
    io = cls_ref.shape[2:]
    cls_ref[0] = y[0:NUM_CLASSES].reshape((NUM_CLASSES,) + io)
    rc = y[NUM_CLASSES:NUM_CLASSES + 8]
    ri = lax.broadcasted_iota(jnp.int32, (8, 1), 0)
    rcp = jnp.where((ri >= 1) & (ri < 5),
                    jnp.maximum(rc * jnp.float32(stride), 0.0), rc)
    cent_ref[0] = rcp[0:1].reshape((1,) + io)
    reg_ref[0] = rcp[1:5].reshape((4,) + io)


def _fused_kernel(x0_ref, xc_ref, w3_ref, pm_ref, wf_ref, fb_ref, *out_refs):
    cls_refs = out_refs[0:NLEV]
    reg_refs = out_refs[NLEV:2 * NLEV]
    cent_refs = out_refs[2 * NLEV:]
    pm = pm_ref[...]
    b3 = jnp.concatenate([pm[:, 0:1], pm[:, 3:4]], axis=0)
    gam = jnp.concatenate([pm[:, 1:2], pm[:, 4:5]], axis=0)
    bet = jnp.concatenate([pm[:, 2:3], pm[:, 5:6]], axis=0)
    params = (b3, gam, bet)
    x0 = x0_ref[0].astype(jnp.bfloat16)
    xc = xc_ref[0]
    for l in range(NLEV):
        H, W = SIZES[l]
        if l == 0:
            x = x0
        else:
            off = CAT_OFF[l - 1]
            x = xc[:, off:off + H * W]
        _do_level(x, w3_ref, params, wf_ref, fb_ref,
                  cls_refs[l], cent_refs[l], reg_refs[l],
                  H, W, STRIDES[l])


def kernel(fpn0, fpn1, fpn2, fpn3, fpn4,
           cls_w, cls_b, cls_g, cls_beta, cls_fw, cls_fb,
           reg_w, reg_b, reg_g, reg_beta, reg_fw, reg_fb):
    f32 = jnp.float32
    B = fpn0.shape[0]
    wtap = jnp.concatenate([cls_w, reg_w], axis=0)
    W3 = jnp.transpose(wtap, (0, 2, 3, 1)).reshape(HID, 9 * IN_CH)
    W3 = W3.astype(jnp.bfloat16)
    pm = jnp.stack([cls_b, cls_g, cls_beta, reg_b, reg_g, reg_beta,
                    jnp.zeros_like(cls_b), jnp.zeros_like(cls_b)], axis=1)
    Wf = jnp.zeros((OUT_ROWS, HID), f32)
    Wf = Wf.at[0:NUM_CLASSES, 0:IN_CH].set(cls_fw[:, :, 0, 0])
    Wf = Wf.at[NUM_CLASSES:NUM_CLASSES + 5, IN_CH:HID].set(reg_fw[:, :, 0, 0])
    Wf = Wf.astype(jnp.bfloat16)
    fb = jnp.concatenate([cls_fb, reg_fb,
                          jnp.zeros((3,), f32)]).reshape(OUT_ROWS, 1)

    x0 = fpn0.reshape(B, IN_CH, 4096)
    bf16 = jnp.bfloat16
    xc = jnp.concatenate(
        [fpn1.reshape(B, IN_CH, 1024).astype(bf16),
         fpn2.reshape(B, IN_CH, 256).astype(bf16),
         fpn3.reshape(B, IN_CH, 64).astype(bf16),
         jnp.zeros((B, IN_CH, 64), bf16),
         fpn4.reshape(B, IN_CH, 16).astype(bf16)], axis=2)
    full = lambda shp: pl.BlockSpec(shp, lambda b: (0,) * len(shp))
    bspec = lambda c, io: pl.BlockSpec((1, c) + io,
                                       lambda b: (b,) + (0,) * (1 + len(io)))
    out_shape = (
        tuple(jax.ShapeDtypeStruct((B, NUM_CLASSES) + io, f32) for io in IO)
        + tuple(jax.ShapeDtypeStruct((B, 4) + io, f32) for io in IO)
        + tuple(jax.ShapeDtypeStruct((B, 1) + io, f32) for io in IO)
    )
    outs = pl.pallas_call(
        _fused_kernel,
        grid=(B,),
        in_specs=(
            [bspec(IN_CH, (4096,)), bspec(IN_CH, (CAT_LEN,)),
             full((HID, 9 * IN_CH)),
             full((IN_CH, 8)), full((OUT_ROWS, HID)), full((OUT_ROWS, 1))]
        ),
        out_specs=(
            tuple(bspec(NUM_CLASSES, io) for io in IO)
            + tuple(bspec(4, io) for io in IO)
            + tuple(bspec(1, io) for io in IO)
        ),
        out_shape=out_shape,
    )(x0, xc, W3, pm, Wf, fb)
    cls_out = [o.reshape(B, NUM_CLASSES, h, w)
               for o, (h, w) in zip(outs[0:NLEV], SIZES)]
    reg_out = [o.reshape(B, 4, h, w)
               for o, (h, w) in zip(outs[NLEV:2 * NLEV], SIZES)]
    cent_out = [o.reshape(B, 1, h, w)
                for o, (h, w) in zip(outs[2 * NLEV:], SIZES)]
    return tuple(cls_out) + tuple(reg_out) + tuple(cent_out)


# speedup vs baseline: 1.1981x; 1.0048x over previous
import jax
import jax.numpy as jnp
import numpy as np
from jax import lax
from jax.experimental import pallas as pl

IN_CH = 96
HID = 192
NUM_CLASSES = 80
OUT_ROWS = 88
GN_EPS = 1e-05
STRIDES = (8, 16, 32, 64, 128)
SIZES = ((64, 64), (32, 32), (16, 16), (8, 8), (4, 4))
NLEV = 5
KSPLIT = 4
CAT_OFF = (0, 1024, 1280, 1408)
CAT_LEN = 1424
IO = ((4096,), (1024,), (256,), (64,), (16,))


def _lane_sum(v, square=False):
    R, S = v.shape
    if S >= 256 and S % 128 == 0:
        chunks = [v[:, 128 * i:128 * (i + 1)] for i in range(S // 128)]
        if square:
            chunks = [c * c for c in chunks]
        while len(chunks) > 1:
            nxt = [chunks[j] + chunks[j + 1]
                   for j in range(0, len(chunks) - 1, 2)]
            if len(chunks) % 2:
                nxt.append(chunks[-1])
            chunks = nxt
        return jnp.sum(chunks[0], axis=1, keepdims=True)
    if square:
        v = v * v
    return jnp.sum(v, axis=1, keepdims=True)


def _do_level(x, w3_ref, params, wf_ref, fb_ref,
              cls_ref, cent_ref, reg_ref, H, W, stride):
    b3, gam, bet = params
    S = H * W
    logw = W.bit_length() - 1
    pos = lax.broadcasted_iota(jnp.int32, (1, S), 1)
    col = pos & (W - 1)
    row = pos >> logw
    row_ok = {dy: (row + dy >= 0) & (row + dy < H) for dy in (-1, 0, 1)}
    col_ok = {dx: (col + dx >= 0) & (col + dx < W) for dx in (-1, 0, 1)}
    parts = []
    for dy in (-1, 0, 1):
        for dx in (-1, 0, 1):
            k = dy * W + dx
            sh = jnp.roll(x, -k, axis=1) if k else x
            parts.append(jnp.where(row_ok[dy] & col_ok[dx], sh,
                                   jnp.bfloat16(0)))
    xcol_a = jnp.concatenate(parts[:KSPLIT], axis=0)
    xcol_b = jnp.concatenate(parts[KSPLIT:], axis=0)
    ka = KSPLIT * IN_CH
    dn = (((1,), (0,)), ((), ()))
    h = lax.dot_general(w3_ref[:, 0:ka], xcol_a, dn,
                        preferred_element_type=jnp.float32)
    h = h + lax.dot_general(w3_ref[:, ka:9 * IN_CH], xcol_b, dn,
                            preferred_element_type=jnp.float32)
    s1 = _lane_sum(h)
    s2 = _lane_sum(h, square=True)
    s1b = s1 + jnp.float32(S) * b3
    s2b = s2 + 2.0 * b3 * s1 + jnp.float32(S) * b3 * b3
    st = jnp.concatenate([s1b, s2b], axis=0)
    ci = lax.broadcasted_iota(jnp.int32, (2 * HID, 1), 0)
    a1 = st + jnp.roll(st, -1, axis=0) + jnp.roll(st, -2, axis=0)
    g0 = jnp.where(ci % 3 == 0, a1, 0.0)
    gs = g0 + jnp.roll(g0, 1, axis=0) + jnp.roll(g0, 2, axis=0)
    cnt = 1.0 / (3.0 * S)
    mean = gs[0:HID] * cnt
    var = gs[HID:2 * HID] * cnt - mean * mean
    inv = lax.rsqrt(var + GN_EPS)
    scale = inv * gam
    shift = bet + (b3 - mean) * scale
    hn = h * scale + shift
    a = hn * (0.5 * jnp.tanh(hn * 0.5) + 0.5)
    y = lax.dot_general(wf_ref[...], a.astype(jnp.bfloat16), dn,
                        preferred_element_type=jnp.float32)
    y = y + fb_ref[...]
    io = cls_ref.shape[2:]
    cls_ref[0] = y[0:NUM_CLASSES].reshape((NUM_CLASSES,) + io)
    rc = y[NUM_CLASSES:NUM_CLASSES + 8]
    ri = lax.broadcasted_iota(jnp.int32, (8, 1), 0)
    rcp = jnp.where((ri >= 1) & (ri < 5),
                    jnp.maximum(rc * jnp.float32(stride), 0.0), rc)
    cent_ref[0] = rcp[0:1].reshape((1,) + io)
    reg_ref[0] = rcp[1:5].reshape((4,) + io)


def _fused_kernel(x0_ref, xc_ref, w3_ref, pm_ref, wf_ref, fb_ref, *out_refs):
    cls_refs = out_refs[0:NLEV]
    reg_refs = out_refs[NLEV:2 * NLEV]
    cent_refs = out_refs[2 * NLEV:]
    pm = pm_ref[...]
    b3 = jnp.concatenate([pm[:, 0:1], pm[:, 3:4]], axis=0)
    gam = jnp.concatenate([pm[:, 1:2], pm[:, 4:5]], axis=0)
    bet = jnp.concatenate([pm[:, 2:3], pm[:, 5:6]], axis=0)
    params = (b3, gam, bet)
    x0 = x0_ref[0].astype(jnp.bfloat16)
    xc = xc_ref[0]
    for l in range(NLEV):
        H, W = SIZES[l]
        if l == 0:
            x = x0
        else:
            off = CAT_OFF[l - 1]
            x = xc[:, off:off + H * W]
        _do_level(x, w3_ref, params, wf_ref, fb_ref,
                  cls_refs[l], cent_refs[l], reg_refs[l],
                  H, W, STRIDES[l])


def kernel(fpn0, fpn1, fpn2, fpn3, fpn4,
           cls_w, cls_b, cls_g, cls_beta, cls_fw, cls_fb,
           reg_w, reg_b, reg_g, reg_beta, reg_fw, reg_fb):
    f32 = jnp.float32
    B = fpn0.shape[0]
    wtap = jnp.concatenate([cls_w, reg_w], axis=0)
    W3 = jnp.transpose(wtap, (0, 2, 3, 1)).reshape(HID, 9 * IN_CH)
    W3 = W3.astype(jnp.bfloat16)
    pm = jnp.stack([cls_b, cls_g, cls_beta, reg_b, reg_g, reg_beta,
                    jnp.zeros_like(cls_b), jnp.zeros_like(cls_b)], axis=1)
    Wf = jnp.zeros((OUT_ROWS, HID), f32)
    Wf = Wf.at[0:NUM_CLASSES, 0:IN_CH].set(cls_fw[:, :, 0, 0])
    Wf = Wf.at[NUM_CLASSES:NUM_CLASSES + 5, IN_CH:HID].set(reg_fw[:, :, 0, 0])
    Wf = Wf.astype(jnp.bfloat16)
    fb = jnp.concatenate([cls_fb, reg_fb,
                          jnp.zeros((3,), f32)]).reshape(OUT_ROWS, 1)

    x0 = fpn0.reshape(B, IN_CH, 4096)
    bf16 = jnp.bfloat16
    xc = jnp.concatenate(
        [fpn1.reshape(B, IN_CH, 1024).astype(bf16),
         fpn2.reshape(B, IN_CH, 256).astype(bf16),
         fpn3.reshape(B, IN_CH, 64).astype(bf16),
         jnp.zeros((B, IN_CH, 64), bf16),
         fpn4.reshape(B, IN_CH, 16).astype(bf16)], axis=2)
    full = lambda shp: pl.BlockSpec(shp, lambda b: (0,) * len(shp))
    bspec = lambda c, io: pl.BlockSpec((1, c) + io,
                                       lambda b: (b,) + (0,) * (1 + len(io)))
    out_shape = (
        tuple(jax.ShapeDtypeStruct((B, NUM_CLASSES) + io, f32) for io in IO)
        + tuple(jax.ShapeDtypeStruct((B, 4) + io, f32) for io in IO)
        + tuple(jax.ShapeDtypeStruct((B, 1) + io, f32) for io in IO)
    )
    outs = pl.pallas_call(
        _fused_kernel,
        grid=(B,),
        in_specs=(
            [bspec(IN_CH, (4096,)), bspec(IN_CH, (CAT_LEN,)),
             full((HID, 9 * IN_CH)),
             full((IN_CH, 8)), full((OUT_ROWS, HID)), full((OUT_ROWS, 1))]
        ),
        out_specs=(
            tuple(bspec(NUM_CLASSES, io) for io in IO)
            + tuple(bspec(4, io) for io in IO)
            + tuple(bspec(1, io) for io in IO)
        ),
        out_shape=out_shape,
    )(x0, xc, W3, pm, Wf, fb)
    cls_out = [o.reshape(B, NUM_CLASSES, h, w)
               for o, (h, w) in zip(outs[0:NLEV], SIZES)]
    reg_out = [o.reshape(B, 4, h, w)
               for o, (h, w) in zip(outs[NLEV:2 * NLEV], SIZES)]
    cent_out = [o.reshape(B, 1, h, w)
                for o, (h, w) in zip(outs[2 * NLEV:], SIZES)]
    return tuple(cls_out) + tuple(reg_out) + tuple(cent_out)
